# baseline (speedup 1.0000x reference)
import functools

import jax
import jax.numpy as jnp
from jax import lax
from jax.experimental import pallas as pl
from jax.experimental.pallas import tpu as pltpu
from jax.experimental.pallas import tpu_sc as plsc

L = 16
NC = 2
NS = 16
NW = NC * NS
RPW = 320
NPAD = NW * RPW
SPT = NPAD // NS
D = 16
ATT = 16
F = 128
CW = F + ATT
GW = D * ATT + D
NEG = -1e30

_SC_PARAMS = pltpu.CompilerParams(
    needs_layout_passes=False, use_tc_tiling_on_sc=False)


def _iota16():
    return lax.iota(jnp.int32, L)


def _scmesh():
    return plsc.VectorSubcoreMesh(core_axis_name="c", subcore_axis_name="s")


def _mm_body(x_ref, wkq_ref, w_ref, kq_ref, nh_ref):
    x = x_ref[...]
    kq_ref[...] = jnp.dot(x, wkq_ref[...], preferred_element_type=jnp.float32)
    nh_ref[...] = jnp.dot(x, w_ref[...], preferred_element_type=jnp.float32)


def _matmuls(x_pad, wkq, w):
    blk = 512
    grid = NPAD // blk
    return pl.pallas_call(
        _mm_body,
        grid=(grid,),
        in_specs=[
            pl.BlockSpec((blk, F), lambda i: (i, 0)),
            pl.BlockSpec((F, 2 * ATT), lambda i: (0, 0)),
            pl.BlockSpec((F, F), lambda i: (0, 0)),
        ],
        out_specs=[
            pl.BlockSpec((blk, 2 * ATT), lambda i: (i, 0)),
            pl.BlockSpec((blk, F), lambda i: (i, 0)),
        ],
        out_shape=[
            jax.ShapeDtypeStruct((NPAD, 2 * ATT), jnp.float32),
            jax.ShapeDtypeStruct((NPAD, F), jnp.float32),
        ],
    )(x_pad, wkq, w)


def _bn_body(n_valid, x_ref, g_ref, b_ref, o_ref):
    x = x_ref[...]
    rows = lax.broadcasted_iota(jnp.int32, x.shape, 0)
    xm = jnp.where(rows < n_valid, x, 0.0)
    s = jnp.sum(xm, axis=0, keepdims=True)
    ss = jnp.sum(xm * xm, axis=0, keepdims=True)
    mean = s / n_valid
    var = ss / n_valid - mean * mean
    inv = lax.rsqrt(var + 1e-5)
    y = g_ref[...] * (x - mean) * inv + b_ref[...]
    o_ref[...] = jnp.maximum(y, 0.0)


def _batchnorm_relu(fh0, gamma, beta, n_valid):
    return pl.pallas_call(
        functools.partial(_bn_body, n_valid),
        out_shape=jax.ShapeDtypeStruct((NPAD, F), jnp.float32),
    )(fh0, gamma.reshape(1, F), beta.reshape(1, F))


GRP = 4
NGRP = RPW // GRP
NB = 2


def _sc1_body(n1, rfl_hbm, key_hbm, comb_hbm, out_hbm,
              comb_shr, rfl_v, key_v, cgs, sgs, ogs,
              scs, sss, sos):
    cid = lax.axis_index("c")
    sid = lax.axis_index("s")
    wid = sid * NC + cid
    base = wid * RPW

    tb = sid * SPT
    pltpu.sync_copy(comb_hbm.at[pl.ds(tb, SPT)], comb_shr.at[pl.ds(tb, SPT)])
    pltpu.sync_copy(rfl_hbm.at[pl.ds(base * D, RPW * D)], rfl_v)
    pltpu.sync_copy(key_hbm.at[pl.ds(base, RPW)], key_v)
    plsc.subcore_barrier()

    def _issue(g, b):
        idx = rfl_v.at[pl.ds(g * (GRP * D), GRP * D)]
        pltpu.async_copy(comb_shr.at[idx], cgs[b], scs[b])
        pltpu.async_copy(comb_shr.at[pl.ds(base + g * GRP, GRP)],
                         sgs[b], sss[b])

    def _wait(g, b):
        idx = rfl_v.at[pl.ds(g * (GRP * D), GRP * D)]
        pltpu.make_async_copy(comb_shr.at[idx], cgs[b], scs[b]).wait()
        pltpu.make_async_copy(comb_shr.at[pl.ds(base + g * GRP, GRP)],
                              sgs[b], sss[b]).wait()

    for b in range(NB):
        _issue(b, b)

    @pl.loop(0, NGRP, step=NB)
    def _group(g):
        for b in range(NB):
            cur = g + b
            _wait(cur, b)
            cg, sg, og = cgs[b], sgs[b], ogs[b]

            @pl.when(cur >= 2)
            def _():
                pltpu.make_async_copy(
                    og, out_hbm.at[pl.ds(base + (cur - 2) * GRP, GRP)],
                    sos[b]).wait()

            @pl.loop(0, GRP)
            def _row(r8):
                row = cur * GRP + r8
                kvec = key_v[row, :]
                kb = [jnp.full((L,), kvec[l]) for l in range(ATT)]
                m16 = r8 * D + _iota16()
                recep = jnp.zeros((L,), jnp.float32)
                for l in range(ATT):
                    col = plsc.load_gather(
                        cg, [m16, jnp.full((L,), F + l, jnp.int32)])
                    recep = recep + col * kb[l]
                rfrow = rfl_v[pl.ds(row * D, D)]
                recep = jnp.where(rfrow == n1, NEG, recep)
                mx = jnp.max(recep)
                e = jnp.exp(recep - mx)
                att = e / jnp.sum(e)
                acc = [sg[r8, pl.ds(c * L, L)] for c in range(F // L)]
                for j in range(D):
                    wb = jnp.full((L,), att[j])
                    for c in range(F // L):
                        acc[c] = acc[c] + wb * cg[r8 * D + j, pl.ds(c * L, L)]
                for c in range(F // L):
                    og[r8, pl.ds(c * L, L)] = acc[c]

            pltpu.async_copy(
                og, out_hbm.at[pl.ds(base + cur * GRP, GRP)], sos[b])

            @pl.when(cur + NB < NGRP)
            def _():
                _issue(cur + NB, b)

    pltpu.make_async_copy(
        ogs[0], out_hbm.at[pl.ds(base + (NGRP - 2) * GRP, GRP)], sos[0]).wait()
    pltpu.make_async_copy(
        ogs[1], out_hbm.at[pl.ds(base + (NGRP - 1) * GRP, GRP)], sos[1]).wait()


def _sc_stage1(rfl, key, comb, n1):
    return pl.kernel(
        functools.partial(_sc1_body, n1),
        out_type=jax.ShapeDtypeStruct((NPAD, F), jnp.float32),
        mesh=_scmesh(),
        compiler_params=_SC_PARAMS,
        scratch_types=[
            pltpu.VMEM_SHARED((NPAD, CW), jnp.float32),
            pltpu.VMEM((RPW * D,), jnp.int32),
            pltpu.VMEM((RPW, ATT), jnp.float32),
            [pltpu.VMEM((GRP * D, CW), jnp.float32) for _ in range(NB)],
            [pltpu.VMEM((GRP, CW), jnp.float32) for _ in range(NB)],
            [pltpu.VMEM((GRP, F), jnp.float32) for _ in range(NB)],
            [pltpu.SemaphoreType.DMA for _ in range(NB)],
            [pltpu.SemaphoreType.DMA for _ in range(NB)],
            [pltpu.SemaphoreType.DMA for _ in range(NB)],
        ],
    )(rfl, key, comb)


NB3 = 4


def _merge16(av, ai, bv, bi):
    rv = lax.rev(bv, (0,))
    ri = lax.rev(bi, (0,))
    mv = jnp.maximum(av, rv)
    mi = jnp.where(av >= rv, ai, ri)
    return plsc.sort_key_val(mv, mi, descending=True)


def _sc3_body(n1, rf_hbm, key_hbm, q_hbm, adj_hbm, out_hbm,
              q_shr, rf_v, key_v, nbs, nqs, oid_v, sas, sqs):
    cid = lax.axis_index("c")
    sid = lax.axis_index("s")
    wid = sid * NC + cid
    base = wid * RPW

    tb = sid * SPT
    pltpu.sync_copy(q_hbm.at[pl.ds(tb, SPT)], q_shr.at[pl.ds(tb, SPT)])
    pltpu.sync_copy(rf_hbm.at[pl.ds(base, RPW)], rf_v)
    pltpu.sync_copy(key_hbm.at[pl.ds(base, RPW)], key_v)
    plsc.subcore_barrier()

    def _issue_adj(r, b):
        pltpu.async_copy(adj_hbm.at[rf_v.at[r]], nbs[b], sas[b])

    def _wait_adj(r, b):
        pltpu.make_async_copy(adj_hbm.at[rf_v.at[r]], nbs[b], sas[b]).wait()

    def _issue_q(b):
        for j in range(D):
            pltpu.async_copy(q_shr.at[nbs[b].at[j]],
                             nqs[b].at[pl.ds(j * D, D)], sqs[b])

    def _wait_q(b):
        for j in range(D):
            pltpu.make_async_copy(q_shr.at[nbs[b].at[j]],
                                  nqs[b].at[pl.ds(j * D, D)], sqs[b]).wait()

    _issue_adj(0, 0)
    _issue_adj(1, 1)
    _wait_adj(0, 0)
    _issue_q(0)

    @pl.loop(0, RPW, step=NB3)
    def _rows(r):
        for b in range(NB3):
            cur = r + b
            _wait_q(b)

            @pl.when(cur + 2 < RPW)
            def _():
                _issue_adj(cur + 2, (b + 2) % NB3)

            @pl.when(cur + 1 < RPW)
            def _():
                _wait_adj(cur + 1, (b + 1) % NB3)
                _issue_q((b + 1) % NB3)

            nq = nqs[b]
            nbr = nbs[b]
            kvec = key_v[cur, :]
            kb = [jnp.full((L,), kvec[l]) for l in range(ATT)]
            pairs = []
            for j in range(D):
                m16 = j * D + _iota16()
                a = jnp.zeros((L,), jnp.float32)
                for l in range(ATT):
                    col = plsc.load_gather(
                        nq, [m16, jnp.full((L,), l, jnp.int32)])
                    a = a + col * kb[l]
                ids = nbr[j, :]
                a = jnp.where(ids == n1, NEG, a)
                pairs.append(plsc.sort_key_val(a, ids, descending=True))

            while len(pairs) > 1:
                nxt = []
                for i in range(0, len(pairs), 2):
                    nxt.append(_merge16(pairs[i][0], pairs[i][1],
                                        pairs[i + 1][0], pairs[i + 1][1]))
                pairs = nxt
            oid_v[cur, :] = pairs[0][1]

    pltpu.sync_copy(oid_v, out_hbm.at[pl.ds(base, RPW)])


def _sc_stage3(rf_pad, key, query, adj_pad, n1):
    return pl.kernel(
        functools.partial(_sc3_body, n1),
        out_type=jax.ShapeDtypeStruct((NPAD, D), jnp.int32),
        mesh=_scmesh(),
        compiler_params=_SC_PARAMS,
        scratch_types=[
            pltpu.VMEM_SHARED((NPAD, ATT), jnp.float32),
            pltpu.VMEM((RPW, D), jnp.int32),
            pltpu.VMEM((RPW, ATT), jnp.float32),
            [pltpu.VMEM((D, D), jnp.int32) for _ in range(NB3)],
            [pltpu.VMEM((D * D, ATT), jnp.float32) for _ in range(NB3)],
            pltpu.VMEM((RPW, D), jnp.int32),
            [pltpu.SemaphoreType.DMA for _ in range(NB3)],
            [pltpu.SemaphoreType.DMA for _ in range(NB3)],
        ],
    )(rf_pad, key, query, adj_pad)


def kernel(input, receptive_field, adj, W, Wk, Wq, bn_gamma, bn_beta):
    n = input.shape[0]
    n1 = n - 1
    x_pad = jnp.pad(input, ((0, NPAD - n), (0, 0)))
    wkq = jnp.concatenate([Wk, Wq], axis=1)
    kq, nh = _matmuls(x_pad, wkq, W[0])
    key = kq[:, :ATT]
    query = kq[:, ATT:]

    rf1 = receptive_field[0]
    rf_pad = jnp.pad(rf1, ((0, NPAD - n), (0, 0)))
    rfl = rf_pad.reshape(-1)
    adj_pad = jnp.pad(adj, ((0, NPAD - n), (0, 0)))

    comb = jnp.concatenate([nh, query], axis=1)
    fh0 = _sc_stage1(rfl, key, comb, n1)
    fh = _batchnorm_relu(fh0, bn_gamma, bn_beta, n)[:n]

    expand = _sc_stage3(rf_pad, key, query, adj_pad, n1)[:n]
    rf_new = jnp.concatenate([receptive_field, expand[None]], axis=0)
    return fh, rf_new


# speedup vs baseline: 1.3679x; 1.3679x over previous
import functools

import jax
import jax.numpy as jnp
from jax import lax
from jax.experimental import pallas as pl
from jax.experimental.pallas import tpu as pltpu
from jax.experimental.pallas import tpu_sc as plsc

L = 16
NC = 2
NS = 16
NW = NC * NS
RPW = 320
NPAD = NW * RPW
SPT = NPAD // NS
D = 16
ATT = 16
F = 128
CW = F + ATT
GW = D * ATT + D
NEG = -1e30

_SC_PARAMS = pltpu.CompilerParams(
    needs_layout_passes=False, use_tc_tiling_on_sc=False)


def _iota16():
    return lax.iota(jnp.int32, L)


def _scmesh():
    return plsc.VectorSubcoreMesh(core_axis_name="c", subcore_axis_name="s")


def _mm_body(x_ref, wkq_ref, w_ref, kq_ref, nh_ref):
    x = x_ref[...]
    kq_ref[...] = jnp.dot(x, wkq_ref[...], preferred_element_type=jnp.float32)
    nh_ref[...] = jnp.dot(x, w_ref[...], preferred_element_type=jnp.float32)


def _matmuls(x_pad, wkq, w):
    blk = 512
    grid = NPAD // blk
    return pl.pallas_call(
        _mm_body,
        grid=(grid,),
        in_specs=[
            pl.BlockSpec((blk, F), lambda i: (i, 0)),
            pl.BlockSpec((F, 2 * ATT), lambda i: (0, 0)),
            pl.BlockSpec((F, F), lambda i: (0, 0)),
        ],
        out_specs=[
            pl.BlockSpec((blk, 2 * ATT), lambda i: (i, 0)),
            pl.BlockSpec((blk, F), lambda i: (i, 0)),
        ],
        out_shape=[
            jax.ShapeDtypeStruct((NPAD, 2 * ATT), jnp.float32),
            jax.ShapeDtypeStruct((NPAD, F), jnp.float32),
        ],
    )(x_pad, wkq, w)


def _bn_body(n_valid, x_ref, g_ref, b_ref, o_ref):
    x = x_ref[...]
    rows = lax.broadcasted_iota(jnp.int32, x.shape, 0)
    xm = jnp.where(rows < n_valid, x, 0.0)
    s = jnp.sum(xm, axis=0, keepdims=True)
    ss = jnp.sum(xm * xm, axis=0, keepdims=True)
    mean = s / n_valid
    var = ss / n_valid - mean * mean
    inv = lax.rsqrt(var + 1e-5)
    y = g_ref[...] * (x - mean) * inv + b_ref[...]
    o_ref[...] = jnp.maximum(y, 0.0)


def _batchnorm_relu(fh0, gamma, beta, n_valid):
    return pl.pallas_call(
        functools.partial(_bn_body, n_valid),
        out_shape=jax.ShapeDtypeStruct((NPAD, F), jnp.float32),
    )(fh0, gamma.reshape(1, F), beta.reshape(1, F))


GRP = 4
NGRP = RPW // GRP
NB = 2


def _sc1_body(n1, rfl_hbm, key_hbm, comb_hbm, out_hbm,
              comb_shr, rfl_v, key_v, cgs, sgs, ogs,
              scs, sss, sos):
    cid = lax.axis_index("c")
    sid = lax.axis_index("s")
    wid = sid * NC + cid
    base = wid * RPW

    tb = sid * SPT
    pltpu.sync_copy(comb_hbm.at[pl.ds(tb, SPT)], comb_shr.at[pl.ds(tb, SPT)])
    pltpu.sync_copy(rfl_hbm.at[pl.ds(base * D, RPW * D)], rfl_v)
    pltpu.sync_copy(key_hbm.at[pl.ds(base, RPW)], key_v)
    plsc.subcore_barrier()

    def _issue(g, b):
        idx = rfl_v.at[pl.ds(g * (GRP * D), GRP * D)]
        pltpu.async_copy(comb_shr.at[idx], cgs[b], scs[b])
        pltpu.async_copy(comb_shr.at[pl.ds(base + g * GRP, GRP)],
                         sgs[b], sss[b])

    def _wait(g, b):
        idx = rfl_v.at[pl.ds(g * (GRP * D), GRP * D)]
        pltpu.make_async_copy(comb_shr.at[idx], cgs[b], scs[b]).wait()
        pltpu.make_async_copy(comb_shr.at[pl.ds(base + g * GRP, GRP)],
                              sgs[b], sss[b]).wait()

    for b in range(NB):
        _issue(b, b)

    @pl.loop(0, NGRP, step=NB)
    def _group(g):
        for b in range(NB):
            cur = g + b
            _wait(cur, b)
            cg, sg, og = cgs[b], sgs[b], ogs[b]

            @pl.when(cur >= 2)
            def _():
                pltpu.make_async_copy(
                    og, out_hbm.at[pl.ds(base + (cur - 2) * GRP, GRP)],
                    sos[b]).wait()

            @pl.loop(0, GRP)
            def _row(r8):
                row = cur * GRP + r8
                kvec = key_v[row, :]
                kb = [jnp.full((L,), kvec[l]) for l in range(ATT)]
                m16 = r8 * D + _iota16()
                recep = jnp.zeros((L,), jnp.float32)
                for l in range(ATT):
                    col = plsc.load_gather(
                        cg, [m16, jnp.full((L,), F + l, jnp.int32)])
                    recep = recep + col * kb[l]
                rfrow = rfl_v[pl.ds(row * D, D)]
                recep = jnp.where(rfrow == n1, NEG, recep)
                mx = jnp.max(recep)
                e = jnp.exp(recep - mx)
                att = e / jnp.sum(e)
                acc = [sg[r8, pl.ds(c * L, L)] for c in range(F // L)]
                for j in range(D):
                    wb = jnp.full((L,), att[j])
                    for c in range(F // L):
                        acc[c] = acc[c] + wb * cg[r8 * D + j, pl.ds(c * L, L)]
                for c in range(F // L):
                    og[r8, pl.ds(c * L, L)] = acc[c]

            pltpu.async_copy(
                og, out_hbm.at[pl.ds(base + cur * GRP, GRP)], sos[b])

            @pl.when(cur + NB < NGRP)
            def _():
                _issue(cur + NB, b)

    pltpu.make_async_copy(
        ogs[0], out_hbm.at[pl.ds(base + (NGRP - 2) * GRP, GRP)], sos[0]).wait()
    pltpu.make_async_copy(
        ogs[1], out_hbm.at[pl.ds(base + (NGRP - 1) * GRP, GRP)], sos[1]).wait()


def _sc_stage1(rfl, key, comb, n1):
    return pl.kernel(
        functools.partial(_sc1_body, n1),
        out_type=jax.ShapeDtypeStruct((NPAD, F), jnp.float32),
        mesh=_scmesh(),
        compiler_params=_SC_PARAMS,
        scratch_types=[
            pltpu.VMEM_SHARED((NPAD, CW), jnp.float32),
            pltpu.VMEM((RPW * D,), jnp.int32),
            pltpu.VMEM((RPW, ATT), jnp.float32),
            [pltpu.VMEM((GRP * D, CW), jnp.float32) for _ in range(NB)],
            [pltpu.VMEM((GRP, CW), jnp.float32) for _ in range(NB)],
            [pltpu.VMEM((GRP, F), jnp.float32) for _ in range(NB)],
            [pltpu.SemaphoreType.DMA for _ in range(NB)],
            [pltpu.SemaphoreType.DMA for _ in range(NB)],
            [pltpu.SemaphoreType.DMA for _ in range(NB)],
        ],
    )(rfl, key, comb)


NBG = 4


def _gb_body(adjf_hbm, q_hbm, g_hbm, adjf_v, qas, gts, ss, sgs):
    wid = lax.axis_index("s") * NC + lax.axis_index("c")
    base = wid * RPW
    pltpu.sync_copy(adjf_hbm.at[pl.ds(base * D, RPW * D)], adjf_v)

    def _issue(g, b):
        idx = adjf_v.at[pl.ds(g * (GRP * D), GRP * D)]
        pltpu.async_copy(q_hbm.at[idx], qas[b], ss[b])

    def _wait(g, b):
        idx = adjf_v.at[pl.ds(g * (GRP * D), GRP * D)]
        pltpu.make_async_copy(q_hbm.at[idx], qas[b], ss[b]).wait()

    for b in range(NBG):
        _issue(b, b)

    @pl.loop(0, NGRP, step=NBG)
    def _group(g):
        for b in range(NBG):
            cur = g + b
            _wait(cur, b)
            qa, gt = qas[b], gts[b % 2]

            @pl.when(cur >= 2)
            def _():
                pltpu.make_async_copy(
                    gt, g_hbm.at[pl.ds(base + (cur - 2) * GRP, GRP)],
                    sgs[b % 2]).wait()

            @pl.loop(0, GRP)
            def _node(r8):
                m16 = r8 * D + _iota16()
                for l in range(ATT):
                    col = plsc.load_gather(
                        qa, [m16, jnp.full((L,), l, jnp.int32)])
                    gt[r8, pl.ds(l * L, L)] = col
                ids = adjf_v[pl.ds((cur * GRP + r8) * D, D)]
                gt[r8, pl.ds(D * ATT, D)] = plsc.bitcast(ids, jnp.float32)

            pltpu.async_copy(
                gt, g_hbm.at[pl.ds(base + cur * GRP, GRP)], sgs[b % 2])

            @pl.when(cur + NBG < NGRP)
            def _():
                _issue(cur + NBG, b)

    pltpu.make_async_copy(
        gts[0], g_hbm.at[pl.ds(base + (NGRP - 2) * GRP, GRP)], sgs[0]).wait()
    pltpu.make_async_copy(
        gts[1], g_hbm.at[pl.ds(base + (NGRP - 1) * GRP, GRP)], sgs[1]).wait()


def _g_build(adjf, query):
    return pl.kernel(
        _gb_body,
        out_type=jax.ShapeDtypeStruct((NPAD, GW), jnp.float32),
        mesh=_scmesh(),
        compiler_params=_SC_PARAMS,
        scratch_types=[
            pltpu.VMEM((RPW * D,), jnp.int32),
            [pltpu.VMEM((GRP * D, ATT), jnp.float32) for _ in range(NBG)],
            [pltpu.VMEM((GRP, GW), jnp.float32) for _ in range(2)],
            [pltpu.SemaphoreType.DMA for _ in range(NBG)],
            [pltpu.SemaphoreType.DMA for _ in range(2)],
        ],
    )(adjf, query)


NB3 = 2


def _merge16(av, ai, bv, bi):
    rv = lax.rev(bv, (0,))
    ri = lax.rev(bi, (0,))
    mv = jnp.maximum(av, rv)
    mi = jnp.where(av >= rv, ai, ri)
    return plsc.sort_key_val(mv, mi, descending=True)


def _sc3_body(n1, rf_hbm, key_hbm, g_hbm, out_hbm,
              rf_v, key_v, grs, oid_v, ss):
    wid = lax.axis_index("s") * NC + lax.axis_index("c")
    base = wid * RPW
    pltpu.sync_copy(rf_hbm.at[pl.ds(base, RPW)], rf_v)
    pltpu.sync_copy(key_hbm.at[pl.ds(base, RPW)], key_v)

    def _issue(r, b):
        pltpu.async_copy(g_hbm.at[rf_v.at[r]], grs[b], ss[b])

    def _wait(r, b):
        pltpu.make_async_copy(g_hbm.at[rf_v.at[r]], grs[b], ss[b]).wait()

    for b in range(NB3):
        _issue(b, b)

    @pl.loop(0, RPW, step=NB3)
    def _rows(r):
        for b in range(NB3):
            cur = r + b
            _wait(cur, b)
            gr = grs[b]
            kvec = key_v[cur, :]
            kb = [jnp.full((L,), kvec[l]) for l in range(ATT)]
            pairs = []
            for j in range(D):
                a = jnp.zeros((L,), jnp.float32)
                for l in range(ATT):
                    a = a + kb[l] * gr[j, pl.ds(l * L, L)]
                ids = plsc.bitcast(gr[j, pl.ds(D * ATT, D)], jnp.int32)
                a = jnp.where(ids == n1, NEG, a)
                pairs.append(plsc.sort_key_val(a, ids, descending=True))

            @pl.when(cur + NB3 < RPW)
            def _():
                _issue(cur + NB3, b)

            while len(pairs) > 1:
                nxt = []
                for i in range(0, len(pairs), 2):
                    nxt.append(_merge16(pairs[i][0], pairs[i][1],
                                        pairs[i + 1][0], pairs[i + 1][1]))
                pairs = nxt
            oid_v[cur, :] = pairs[0][1]

    pltpu.sync_copy(oid_v, out_hbm.at[pl.ds(base, RPW)])


def _sc_stage3(rf_pad, key, g_tab, n1):
    return pl.kernel(
        functools.partial(_sc3_body, n1),
        out_type=jax.ShapeDtypeStruct((NPAD, D), jnp.int32),
        mesh=_scmesh(),
        compiler_params=_SC_PARAMS,
        scratch_types=[
            pltpu.VMEM((RPW, D), jnp.int32),
            pltpu.VMEM((RPW, ATT), jnp.float32),
            [pltpu.VMEM((D, GW), jnp.float32) for _ in range(NB3)],
            pltpu.VMEM((RPW, D), jnp.int32),
            [pltpu.SemaphoreType.DMA for _ in range(NB3)],
        ],
    )(rf_pad, key, g_tab)


def kernel(input, receptive_field, adj, W, Wk, Wq, bn_gamma, bn_beta):
    n = input.shape[0]
    n1 = n - 1
    x_pad = jnp.pad(input, ((0, NPAD - n), (0, 0)))
    wkq = jnp.concatenate([Wk, Wq], axis=1)
    kq, nh = _matmuls(x_pad, wkq, W[0])
    key = kq[:, :ATT]
    query = kq[:, ATT:]

    rf1 = receptive_field[0]
    rf_pad = jnp.pad(rf1, ((0, NPAD - n), (0, 0)))
    rfl = rf_pad.reshape(-1)
    adjf = jnp.pad(adj, ((0, NPAD - n), (0, 0))).reshape(-1)

    comb = jnp.concatenate([nh, query], axis=1)
    fh0 = _sc_stage1(rfl, key, comb, n1)
    fh = _batchnorm_relu(fh0, bn_gamma, bn_beta, n)[:n]

    g_tab = _g_build(adjf, query)
    expand = _sc_stage3(rf_pad, key, g_tab, n1)[:n]
    rf_new = jnp.concatenate([receptive_field, expand[None]], axis=0)
    return fh, rf_new


# speedup vs baseline: 1.3862x; 1.0134x over previous
import functools

import jax
import jax.numpy as jnp
from jax import lax
from jax.experimental import pallas as pl
from jax.experimental.pallas import tpu as pltpu
from jax.experimental.pallas import tpu_sc as plsc

L = 16
NC = 2
NS = 16
NW = NC * NS
RPW = 320
NPAD = NW * RPW
SPT = NPAD // NS
D = 16
ATT = 16
F = 128
CW = F + ATT
GW = D * ATT + D
NEG = -1e30

_SC_PARAMS = pltpu.CompilerParams(
    needs_layout_passes=False, use_tc_tiling_on_sc=False)


def _iota16():
    return lax.iota(jnp.int32, L)


def _scmesh():
    return plsc.VectorSubcoreMesh(core_axis_name="c", subcore_axis_name="s")


def _mm_body(x_ref, wkq_ref, w_ref, kq_ref, nh_ref):
    x = x_ref[...]
    kq_ref[...] = jnp.dot(x, wkq_ref[...], preferred_element_type=jnp.float32)
    nh_ref[...] = jnp.dot(x, w_ref[...], preferred_element_type=jnp.float32)


def _matmuls(x_pad, wkq, w):
    blk = 512
    grid = NPAD // blk
    return pl.pallas_call(
        _mm_body,
        grid=(grid,),
        in_specs=[
            pl.BlockSpec((blk, F), lambda i: (i, 0)),
            pl.BlockSpec((F, 2 * ATT), lambda i: (0, 0)),
            pl.BlockSpec((F, F), lambda i: (0, 0)),
        ],
        out_specs=[
            pl.BlockSpec((blk, 2 * ATT), lambda i: (i, 0)),
            pl.BlockSpec((blk, F), lambda i: (i, 0)),
        ],
        out_shape=[
            jax.ShapeDtypeStruct((NPAD, 2 * ATT), jnp.float32),
            jax.ShapeDtypeStruct((NPAD, F), jnp.float32),
        ],
    )(x_pad, wkq, w)


def _bn_body(n_valid, x_ref, g_ref, b_ref, o_ref):
    x = x_ref[...]
    rows = lax.broadcasted_iota(jnp.int32, x.shape, 0)
    xm = jnp.where(rows < n_valid, x, 0.0)
    s = jnp.sum(xm, axis=0, keepdims=True)
    ss = jnp.sum(xm * xm, axis=0, keepdims=True)
    mean = s / n_valid
    var = ss / n_valid - mean * mean
    inv = lax.rsqrt(var + 1e-5)
    y = g_ref[...] * (x - mean) * inv + b_ref[...]
    o_ref[...] = jnp.maximum(y, 0.0)


def _batchnorm_relu(fh0, gamma, beta, n_valid):
    return pl.pallas_call(
        functools.partial(_bn_body, n_valid),
        out_shape=jax.ShapeDtypeStruct((NPAD, F), jnp.float32),
    )(fh0, gamma.reshape(1, F), beta.reshape(1, F))


GRP = 4
NGRP = RPW // GRP
NB = 2


def _sc1_body(n1, rfl_hbm, key_hbm, comb_hbm, out_hbm,
              comb_shr, rfl_v, key_v, cgs, sgs, ogs,
              scs, sss, sos):
    cid = lax.axis_index("c")
    sid = lax.axis_index("s")
    wid = sid * NC + cid
    base = wid * RPW

    tb = sid * SPT
    pltpu.sync_copy(comb_hbm.at[pl.ds(tb, SPT)], comb_shr.at[pl.ds(tb, SPT)])
    pltpu.sync_copy(rfl_hbm.at[pl.ds(base * D, RPW * D)], rfl_v)
    pltpu.sync_copy(key_hbm.at[pl.ds(base, RPW)], key_v)
    plsc.subcore_barrier()

    def _issue(g, b):
        idx = rfl_v.at[pl.ds(g * (GRP * D), GRP * D)]
        pltpu.async_copy(comb_shr.at[idx], cgs[b], scs[b])
        pltpu.async_copy(comb_shr.at[pl.ds(base + g * GRP, GRP)],
                         sgs[b], sss[b])

    def _wait(g, b):
        idx = rfl_v.at[pl.ds(g * (GRP * D), GRP * D)]
        pltpu.make_async_copy(comb_shr.at[idx], cgs[b], scs[b]).wait()
        pltpu.make_async_copy(comb_shr.at[pl.ds(base + g * GRP, GRP)],
                              sgs[b], sss[b]).wait()

    for b in range(NB):
        _issue(b, b)

    @pl.loop(0, NGRP, step=NB)
    def _group(g):
        for b in range(NB):
            cur = g + b
            _wait(cur, b)
            cg, sg, og = cgs[b], sgs[b], ogs[b]

            @pl.when(cur >= 2)
            def _():
                pltpu.make_async_copy(
                    og, out_hbm.at[pl.ds(base + (cur - 2) * GRP, GRP)],
                    sos[b]).wait()

            @pl.loop(0, GRP)
            def _row(r8):
                row = cur * GRP + r8
                kvec = key_v[row, :]
                kb = [jnp.full((L,), kvec[l]) for l in range(ATT)]
                m16 = r8 * D + _iota16()
                recep = jnp.zeros((L,), jnp.float32)
                for l in range(ATT):
                    col = plsc.load_gather(
                        cg, [m16, jnp.full((L,), F + l, jnp.int32)])
                    recep = recep + col * kb[l]
                rfrow = rfl_v[pl.ds(row * D, D)]
                recep = jnp.where(rfrow == n1, NEG, recep)
                mx = jnp.max(recep)
                e = jnp.exp(recep - mx)
                att = e / jnp.sum(e)
                acc = [sg[r8, pl.ds(c * L, L)] for c in range(F // L)]
                for j in range(D):
                    wb = jnp.full((L,), att[j])
                    for c in range(F // L):
                        acc[c] = acc[c] + wb * cg[r8 * D + j, pl.ds(c * L, L)]
                for c in range(F // L):
                    og[r8, pl.ds(c * L, L)] = acc[c]

            pltpu.async_copy(
                og, out_hbm.at[pl.ds(base + cur * GRP, GRP)], sos[b])

            @pl.when(cur + NB < NGRP)
            def _():
                _issue(cur + NB, b)

    pltpu.make_async_copy(
        ogs[0], out_hbm.at[pl.ds(base + (NGRP - 2) * GRP, GRP)], sos[0]).wait()
    pltpu.make_async_copy(
        ogs[1], out_hbm.at[pl.ds(base + (NGRP - 1) * GRP, GRP)], sos[1]).wait()


def _sc_stage1(rfl, key, comb, n1):
    return pl.kernel(
        functools.partial(_sc1_body, n1),
        out_type=jax.ShapeDtypeStruct((NPAD, F), jnp.float32),
        mesh=_scmesh(),
        compiler_params=_SC_PARAMS,
        scratch_types=[
            pltpu.VMEM_SHARED((NPAD, CW), jnp.float32),
            pltpu.VMEM((RPW * D,), jnp.int32),
            pltpu.VMEM((RPW, ATT), jnp.float32),
            [pltpu.VMEM((GRP * D, CW), jnp.float32) for _ in range(NB)],
            [pltpu.VMEM((GRP, CW), jnp.float32) for _ in range(NB)],
            [pltpu.VMEM((GRP, F), jnp.float32) for _ in range(NB)],
            [pltpu.SemaphoreType.DMA for _ in range(NB)],
            [pltpu.SemaphoreType.DMA for _ in range(NB)],
            [pltpu.SemaphoreType.DMA for _ in range(NB)],
        ],
    )(rfl, key, comb)


NBG = 4


def _gb_body(adjf_hbm, q_hbm, g_hbm, adjf_v, qas, gts, ss, sgs):
    wid = lax.axis_index("s") * NC + lax.axis_index("c")
    base = wid * RPW
    pltpu.sync_copy(adjf_hbm.at[pl.ds(base * D, RPW * D)], adjf_v)

    def _issue(g, b):
        idx = adjf_v.at[pl.ds(g * (GRP * D), GRP * D)]
        pltpu.async_copy(q_hbm.at[idx], qas[b], ss[b])

    def _wait(g, b):
        idx = adjf_v.at[pl.ds(g * (GRP * D), GRP * D)]
        pltpu.make_async_copy(q_hbm.at[idx], qas[b], ss[b]).wait()

    for b in range(NBG):
        _issue(b, b)

    @pl.loop(0, NGRP, step=NBG)
    def _group(g):
        for b in range(NBG):
            cur = g + b
            _wait(cur, b)
            qa, gt = qas[b], gts[b % 2]

            @pl.when(cur >= 2)
            def _():
                pltpu.make_async_copy(
                    gt, g_hbm.at[pl.ds(base + (cur - 2) * GRP, GRP)],
                    sgs[b % 2]).wait()

            @pl.loop(0, GRP)
            def _node(r8):
                m16 = r8 * D + _iota16()
                for l in range(ATT):
                    col = plsc.load_gather(
                        qa, [m16, jnp.full((L,), l, jnp.int32)])
                    gt[r8, pl.ds(l * L, L)] = col
                ids = adjf_v[pl.ds((cur * GRP + r8) * D, D)]
                gt[r8, pl.ds(D * ATT, D)] = plsc.bitcast(ids, jnp.float32)

            pltpu.async_copy(
                gt, g_hbm.at[pl.ds(base + cur * GRP, GRP)], sgs[b % 2])

            @pl.when(cur + NBG < NGRP)
            def _():
                _issue(cur + NBG, b)

    pltpu.make_async_copy(
        gts[0], g_hbm.at[pl.ds(base + (NGRP - 2) * GRP, GRP)], sgs[0]).wait()
    pltpu.make_async_copy(
        gts[1], g_hbm.at[pl.ds(base + (NGRP - 1) * GRP, GRP)], sgs[1]).wait()


def _g_build(adjf, query):
    return pl.kernel(
        _gb_body,
        out_type=jax.ShapeDtypeStruct((NPAD, GW), jnp.float32),
        mesh=_scmesh(),
        compiler_params=_SC_PARAMS,
        scratch_types=[
            pltpu.VMEM((RPW * D,), jnp.int32),
            [pltpu.VMEM((GRP * D, ATT), jnp.float32) for _ in range(NBG)],
            [pltpu.VMEM((GRP, GW), jnp.float32) for _ in range(2)],
            [pltpu.SemaphoreType.DMA for _ in range(NBG)],
            [pltpu.SemaphoreType.DMA for _ in range(2)],
        ],
    )(adjf, query)


NB3 = 4


def _merge16(av, ai, bv, bi):
    rv = lax.rev(bv, (0,))
    ri = lax.rev(bi, (0,))
    mv = jnp.maximum(av, rv)
    mi = jnp.where(av >= rv, ai, ri)
    return plsc.sort_key_val(mv, mi, descending=True)


def _sc3_body(n1, rf_hbm, key_hbm, g_hbm, out_hbm,
              rf_v, key_v, grs, oid_v, ss):
    wid = lax.axis_index("s") * NC + lax.axis_index("c")
    base = wid * RPW
    pltpu.sync_copy(rf_hbm.at[pl.ds(base, RPW)], rf_v)
    pltpu.sync_copy(key_hbm.at[pl.ds(base, RPW)], key_v)

    def _issue(r, b):
        pltpu.async_copy(g_hbm.at[rf_v.at[r]], grs[b], ss[b])

    def _wait(r, b):
        pltpu.make_async_copy(g_hbm.at[rf_v.at[r]], grs[b], ss[b]).wait()

    for b in range(NB3):
        _issue(b, b)

    @pl.loop(0, RPW, step=NB3)
    def _rows(r):
        for b in range(NB3):
            cur = r + b
            _wait(cur, b)
            gr = grs[b]
            kvec = key_v[cur, :]
            kb = [jnp.full((L,), kvec[l]) for l in range(ATT)]
            pairs = []
            for j in range(D):
                a = jnp.zeros((L,), jnp.float32)
                for l in range(ATT):
                    a = a + kb[l] * gr[j, pl.ds(l * L, L)]
                ids = plsc.bitcast(gr[j, pl.ds(D * ATT, D)], jnp.int32)
                a = jnp.where(ids == n1, NEG, a)
                pairs.append(plsc.sort_key_val(a, ids, descending=True))

            @pl.when(cur + NB3 < RPW)
            def _():
                _issue(cur + NB3, b)

            while len(pairs) > 1:
                nxt = []
                for i in range(0, len(pairs), 2):
                    nxt.append(_merge16(pairs[i][0], pairs[i][1],
                                        pairs[i + 1][0], pairs[i + 1][1]))
                pairs = nxt
            oid_v[cur, :] = pairs[0][1]

    pltpu.sync_copy(oid_v, out_hbm.at[pl.ds(base, RPW)])


def _sc_stage3(rf_pad, key, g_tab, n1):
    return pl.kernel(
        functools.partial(_sc3_body, n1),
        out_type=jax.ShapeDtypeStruct((NPAD, D), jnp.int32),
        mesh=_scmesh(),
        compiler_params=_SC_PARAMS,
        scratch_types=[
            pltpu.VMEM((RPW, D), jnp.int32),
            pltpu.VMEM((RPW, ATT), jnp.float32),
            [pltpu.VMEM((D, GW), jnp.float32) for _ in range(NB3)],
            pltpu.VMEM((RPW, D), jnp.int32),
            [pltpu.SemaphoreType.DMA for _ in range(NB3)],
        ],
    )(rf_pad, key, g_tab)


def kernel(input, receptive_field, adj, W, Wk, Wq, bn_gamma, bn_beta):
    n = input.shape[0]
    n1 = n - 1
    x_pad = jnp.pad(input, ((0, NPAD - n), (0, 0)))
    wkq = jnp.concatenate([Wk, Wq], axis=1)
    kq, nh = _matmuls(x_pad, wkq, W[0])
    key = kq[:, :ATT]
    query = kq[:, ATT:]

    rf1 = receptive_field[0]
    rf_pad = jnp.pad(rf1, ((0, NPAD - n), (0, 0)))
    rfl = rf_pad.reshape(-1)
    adjf = jnp.pad(adj, ((0, NPAD - n), (0, 0))).reshape(-1)

    comb = jnp.concatenate([nh, query], axis=1)
    fh0 = _sc_stage1(rfl, key, comb, n1)
    fh = _batchnorm_relu(fh0, bn_gamma, bn_beta, n)[:n]

    g_tab = _g_build(adjf, query)
    expand = _sc_stage3(rf_pad, key, g_tab, n1)[:n]
    rf_new = jnp.concatenate([receptive_field, expand[None]], axis=0)
    return fh, rf_new
